# Initial kernel scaffold; baseline (speedup 1.0000x reference)
import jax
import jax.numpy as jnp
from jax.experimental import pallas as pl


def kernel(inputs, table):
    raise NotImplementedError("write your pallas kernel here")


# speedup vs baseline: 1.1031x; 1.1031x over previous
import functools

import jax
import jax.numpy as jnp
from jax import lax
from jax.experimental import pallas as pl
from jax.experimental.pallas import tpu as pltpu
from jax.experimental.pallas import tpu_sc as plsc

VOCAB = 1000000
EMBED_DIM = 32
B = 16384
L = 50

NUM_CORES = 2
NUM_SUBCORES = 16
NW = NUM_CORES * NUM_SUBCORES

TOTAL = B * L
B_PER_W = TOTAL // NW
CHUNK = 1600
N_CHUNKS = B_PER_W // CHUNK


def _gather_body(table_hbm, idx_hbm, out_hbm, idx_v, rows_v, sem):
    wid = lax.axis_index("s") * NUM_CORES + lax.axis_index("c")
    base = wid * B_PER_W
    for c in range(N_CHUNKS):
        off = base + c * CHUNK
        pltpu.sync_copy(idx_hbm.at[pl.ds(off, CHUNK)], idx_v)
        pltpu.async_copy(table_hbm.at[idx_v], rows_v, sem).wait()
        pltpu.sync_copy(rows_v, out_hbm.at[pl.ds(off, CHUNK)])


_mesh = plsc.VectorSubcoreMesh(
    core_axis_name="c", subcore_axis_name="s",
    num_cores=NUM_CORES, num_subcores=NUM_SUBCORES,
)

_sc_gather = pl.kernel(
    _gather_body,
    out_type=jax.ShapeDtypeStruct((TOTAL, EMBED_DIM), jnp.float32),
    mesh=_mesh,
    scratch_types=[
        pltpu.VMEM((CHUNK,), jnp.int32),
        pltpu.VMEM((CHUNK, EMBED_DIM), jnp.float32),
        pltpu.SemaphoreType.DMA,
    ],
    compiler_params=pltpu.CompilerParams(use_tc_tiling_on_sc=False),
)


@jax.jit
def kernel(inputs, table):
    idx = inputs.reshape(-1).astype(jnp.int32)
    out = _sc_gather(table, idx)
    return out.reshape(inputs.shape + (EMBED_DIM,))


# speedup vs baseline: 1.1097x; 1.0060x over previous
import functools

import jax
import jax.numpy as jnp
from jax import lax
from jax.experimental import pallas as pl
from jax.experimental.pallas import tpu as pltpu
from jax.experimental.pallas import tpu_sc as plsc

VOCAB = 1000000
EMBED_DIM = 32
B = 16384
L = 50

NUM_CORES = 2
NUM_SUBCORES = 16
NW = NUM_CORES * NUM_SUBCORES

TOTAL = B * L
B_PER_W = TOTAL // NW
CHUNK = 1600
N_CHUNKS = B_PER_W // CHUNK


def _gather_body(table_hbm, idx_hbm, out_hbm, idx_all, rows0, rows1,
                 sem_g0, sem_g1, sem_s0, sem_s1):
    wid = lax.axis_index("s") * NUM_CORES + lax.axis_index("c")
    base = wid * B_PER_W
    rows = (rows0, rows1)
    sg = (sem_g0, sem_g1)
    ss = (sem_s0, sem_s1)

    pltpu.sync_copy(idx_hbm.at[pl.ds(base, B_PER_W)], idx_all)

    gathers = [None] * N_CHUNKS
    stores = [None] * N_CHUNKS
    gathers[0] = pltpu.async_copy(
        table_hbm.at[idx_all.at[pl.ds(0, CHUNK)]], rows[0], sg[0])
    for i in range(N_CHUNKS):
        b = i & 1
        gathers[i].wait()
        stores[i] = pltpu.async_copy(
            rows[b], out_hbm.at[pl.ds(base + i * CHUNK, CHUNK)], ss[b])
        if i + 1 < N_CHUNKS:
            nb = (i + 1) & 1
            if i >= 1:
                stores[i - 1].wait()
            gathers[i + 1] = pltpu.async_copy(
                table_hbm.at[idx_all.at[pl.ds((i + 1) * CHUNK, CHUNK)]],
                rows[nb], sg[nb])
    stores[N_CHUNKS - 1].wait()


_mesh = plsc.VectorSubcoreMesh(
    core_axis_name="c", subcore_axis_name="s",
    num_cores=NUM_CORES, num_subcores=NUM_SUBCORES,
)

_sc_gather = pl.kernel(
    _gather_body,
    out_type=jax.ShapeDtypeStruct((TOTAL, EMBED_DIM), jnp.float32),
    mesh=_mesh,
    scratch_types=[
        pltpu.VMEM((B_PER_W,), jnp.int32),
        pltpu.VMEM((CHUNK, EMBED_DIM), jnp.float32),
        pltpu.VMEM((CHUNK, EMBED_DIM), jnp.float32),
        pltpu.SemaphoreType.DMA,
        pltpu.SemaphoreType.DMA,
        pltpu.SemaphoreType.DMA,
        pltpu.SemaphoreType.DMA,
    ],
    compiler_params=pltpu.CompilerParams(use_tc_tiling_on_sc=False),
)


@jax.jit
def kernel(inputs, table):
    idx = inputs.reshape(-1).astype(jnp.int32)
    out = _sc_gather(table, idx)
    return out.reshape(inputs.shape + (EMBED_DIM,))


# speedup vs baseline: 1.7868x; 1.6101x over previous
import jax
import jax.numpy as jnp
from jax import lax
from jax.experimental import pallas as pl
from jax.experimental.pallas import tpu as pltpu
from jax.experimental.pallas import tpu_sc as plsc

VOCAB = 1000000
EMBED_DIM = 32
B = 16384
L = 50

NUM_CORES = 2
NUM_SUBCORES = 16
NW = NUM_CORES * NUM_SUBCORES

ROWS_PER_W = B // NW
TOTAL_PER_W = ROWS_PER_W * L
CHUNK_ROWS = 32
CHUNK = CHUNK_ROWS * L
N_CHUNKS = ROWS_PER_W // CHUNK_ROWS


def _fire_gather(table_hbm, idx_v, rows, c, sem):
    return pltpu.async_copy(
        table_hbm.at[idx_v.at[pl.ds(c * CHUNK, CHUNK)]], rows, sem)


def _fire_stores(out_hbm, rows, rb0, c, sem):
    return [
        pltpu.async_copy(
            rows.at[pl.ds(L * j, L)], out_hbm.at[rb0 + c * CHUNK_ROWS + j], sem)
        for j in range(CHUNK_ROWS)
    ]


def _gather_body(table_hbm, idx_hbm, out_hbm, idx_v, rows0, rows1,
                 sem_g0, sem_g1, sem_s0, sem_s1):
    wid = lax.axis_index("s") * NUM_CORES + lax.axis_index("c")
    rb0 = wid * ROWS_PER_W
    rows = (rows0, rows1)
    sg = (sem_g0, sem_g1)
    ss = (sem_s0, sem_s1)

    pltpu.sync_copy(idx_hbm.at[pl.ds(wid * TOTAL_PER_W, TOTAL_PER_W)], idx_v)

    stores = [None, None]
    for c in range(N_CHUNKS):
        b = c & 1
        if stores[b] is not None:
            for d in stores[b]:
                d.wait()
        g = _fire_gather(table_hbm, idx_v, rows[b], c, sg[b])
        g.wait()
        stores[b] = _fire_stores(out_hbm, rows[b], rb0, c, ss[b])
    for b in range(2):
        for d in stores[b]:
            d.wait()


_mesh = plsc.VectorSubcoreMesh(
    core_axis_name="c", subcore_axis_name="s",
    num_cores=NUM_CORES, num_subcores=NUM_SUBCORES,
)

_sc_gather = pl.kernel(
    _gather_body,
    out_type=jax.ShapeDtypeStruct((B, L, EMBED_DIM), jnp.float32),
    mesh=_mesh,
    scratch_types=[
        pltpu.VMEM((TOTAL_PER_W,), jnp.int32),
        pltpu.VMEM((CHUNK, EMBED_DIM), jnp.float32),
        pltpu.VMEM((CHUNK, EMBED_DIM), jnp.float32),
        pltpu.SemaphoreType.DMA,
        pltpu.SemaphoreType.DMA,
        pltpu.SemaphoreType.DMA,
        pltpu.SemaphoreType.DMA,
    ],
    compiler_params=pltpu.CompilerParams(use_tc_tiling_on_sc=False),
)


@jax.jit
def kernel(inputs, table):
    idx = inputs.reshape(-1)
    return _sc_gather(table, idx)


# speedup vs baseline: 1.7928x; 1.0034x over previous
import jax
import jax.numpy as jnp
from jax import lax
from jax.experimental import pallas as pl
from jax.experimental.pallas import tpu as pltpu
from jax.experimental.pallas import tpu_sc as plsc

VOCAB = 1000000
EMBED_DIM = 32
B = 16384
L = 50

NUM_CORES = 2
NUM_SUBCORES = 16
NW = NUM_CORES * NUM_SUBCORES

ROWS_PER_W = B // NW
TOTAL_PER_W = ROWS_PER_W * L
CHUNK_ROWS = 32
CHUNK = CHUNK_ROWS * L
N_CHUNKS = ROWS_PER_W // CHUNK_ROWS


def _gather_body(table_hbm, idx_hbm, out_hbm, idx_v, rows0, rows1,
                 sem_g0, sem_g1, sem_s0, sem_s1):
    wid = lax.axis_index("s") * NUM_CORES + lax.axis_index("c")
    rb0 = wid * ROWS_PER_W
    rows = (rows0, rows1)
    sg = (sem_g0, sem_g1)
    ss = (sem_s0, sem_s1)

    pltpu.sync_copy(idx_hbm.at[pl.ds(wid * TOTAL_PER_W, TOTAL_PER_W)], idx_v)

    @pl.loop(0, N_CHUNKS, step=2)
    def _chunks(c):
        for b in range(2):
            cc = c + b

            @pl.when(cc >= 2)
            def _drain():
                pltpu.make_async_copy(
                    table_hbm.at[pl.ds(0, CHUNK)], rows[b], ss[b]).wait()

            gather = pltpu.async_copy(
                table_hbm.at[idx_v.at[pl.ds(cc * CHUNK, CHUNK)]],
                rows[b], sg[b])
            gather.wait()
            for j in range(CHUNK_ROWS):
                pltpu.async_copy(
                    rows[b].at[pl.ds(L * j, L)],
                    out_hbm.at[rb0 + cc * CHUNK_ROWS + j], ss[b])

    for b in range(2):
        pltpu.make_async_copy(
            table_hbm.at[pl.ds(0, CHUNK)], rows[b], ss[b]).wait()


_mesh = plsc.VectorSubcoreMesh(
    core_axis_name="c", subcore_axis_name="s",
    num_cores=NUM_CORES, num_subcores=NUM_SUBCORES,
)

_sc_gather = pl.kernel(
    _gather_body,
    out_type=jax.ShapeDtypeStruct((B, L, EMBED_DIM), jnp.float32),
    mesh=_mesh,
    scratch_types=[
        pltpu.VMEM((TOTAL_PER_W,), jnp.int32),
        pltpu.VMEM((CHUNK, EMBED_DIM), jnp.float32),
        pltpu.VMEM((CHUNK, EMBED_DIM), jnp.float32),
        pltpu.SemaphoreType.DMA,
        pltpu.SemaphoreType.DMA,
        pltpu.SemaphoreType.DMA,
        pltpu.SemaphoreType.DMA,
    ],
    compiler_params=pltpu.CompilerParams(use_tc_tiling_on_sc=False),
)


@jax.jit
def kernel(inputs, table):
    idx = inputs.reshape(-1)
    return _sc_gather(table, idx)
